# Initial kernel scaffold; baseline (speedup 1.0000x reference)
import functools

import jax
import jax.numpy as jnp
from jax import lax
from jax.experimental import pallas as pl
from jax.experimental.pallas import tpu as pltpu
from jax.experimental.pallas import tpu_sc as plsc

_N, _E, _L, _R, _T, _D, _H = 10000, 320000, 2, 4, 3, 128, 128
_TH = _T * _H
_BN = 1000
_NB = _N // _BN
_NTILES = 32
_SB = 128
_NB_E = 79
_EPT = _NB_E * _SB
_EPAD = _NTILES * _EPT
_EROWS = _EPAD // _SB
_NPAD = 10016
_RPT = _NPAD // 16


def _prep_body(w2_ref, bst_ref, b2_ref, c_ref, dvec_ref):
    b = bst_ref[0, 0]
    c_ref[0, 0] = jnp.dot(w2_ref[0, 0], b, preferred_element_type=jnp.float32)
    dvec_ref[0, 0] = jnp.dot(b2_ref[0, 0], b, preferred_element_type=jnp.float32)


def _prep(rel_W2, Bstack, rel_b2r):
    return pl.pallas_call(
        _prep_body,
        grid=(_L, _R),
        in_specs=[
            pl.BlockSpec((1, 1, _H, _H), lambda l, r: (l, r, 0, 0)),
            pl.BlockSpec((1, 1, _D, _TH), lambda l, r: (l, r, 0, 0)),
            pl.BlockSpec((1, 1, 1, _H), lambda l, r: (l, r, 0, 0)),
        ],
        out_specs=[
            pl.BlockSpec((1, 1, _H, _TH), lambda l, r: (l, r, 0, 0)),
            pl.BlockSpec((1, 1, 1, _TH), lambda l, r: (l, r, 0, 0)),
        ],
        out_shape=[
            jax.ShapeDtypeStruct((_L, _R, _H, _TH), jnp.float32),
            jax.ShapeDtypeStruct((_L, _R, 1, _TH), jnp.float32),
        ],
    )(rel_W2, Bstack, rel_b2r)


def _table_body(h_ref, w1_ref, b1_ref, c_ref, dvec_ref, out_ref):
    p = jax.nn.relu(
        jnp.dot(h_ref[...], w1_ref[0], preferred_element_type=jnp.float32)
        + b1_ref[0]
    )
    out_ref[0] = (
        jnp.dot(p, c_ref[0], preferred_element_type=jnp.float32) + dvec_ref[0]
    )


def _table(h, W1l, b1ll, Cl, dvecl):
    return pl.pallas_call(
        _table_body,
        grid=(_R, _NB),
        in_specs=[
            pl.BlockSpec((_BN, _D), lambda r, nb: (nb, 0)),
            pl.BlockSpec((1, _D, _H), lambda r, nb: (r, 0, 0)),
            pl.BlockSpec((1, 1, _H), lambda r, nb: (r, 0, 0)),
            pl.BlockSpec((1, _H, _TH), lambda r, nb: (r, 0, 0)),
            pl.BlockSpec((1, 1, _TH), lambda r, nb: (r, 0, 0)),
        ],
        out_specs=pl.BlockSpec((1, _BN, _TH), lambda r, nb: (r, nb, 0)),
        out_shape=jax.ShapeDtypeStruct((_R, _N, _TH), jnp.float32),
    )(h, W1l, b1ll, Cl, dvecl)


def _update_body(pre_ref, h_ref, nt_ref, a_ref, b1_ref, w2_ref, b2_ref, out_ref):
    s = pre_ref[0] + pre_ref[1]
    y = (
        jnp.dot(h_ref[...], a_ref[...], preferred_element_type=jnp.float32)
        + b1_ref[...]
    )
    nt = nt_ref[...]
    parts = []
    for t in range(_T):
        m = (nt == t).astype(jnp.float32)
        zt = jax.nn.relu(y[:, t * _H:(t + 1) * _H] + s)
        parts.append(zt * m)
    zcat = jnp.concatenate(parts, axis=1)
    out = jnp.dot(zcat, w2_ref[...], preferred_element_type=jnp.float32)
    for t in range(_T):
        m = (nt == t).astype(jnp.float32)
        out = out + m * b2_ref[t][None, :]
    out_ref[...] = out


def _update(pre, h, nt2d, Astackl, b1ll, W2stackl, b2l):
    return pl.pallas_call(
        _update_body,
        grid=(_NB,),
        in_specs=[
            pl.BlockSpec((2, _BN, _H), lambda nb: (0, nb, 0)),
            pl.BlockSpec((_BN, _D), lambda nb: (nb, 0)),
            pl.BlockSpec((_BN, 1), lambda nb: (nb, 0)),
            pl.BlockSpec((_D, _TH), lambda nb: (0, 0)),
            pl.BlockSpec((1, _TH), lambda nb: (0, 0)),
            pl.BlockSpec((_TH, _D), lambda nb: (0, 0)),
            pl.BlockSpec((_T, _H), lambda nb: (0, 0)),
        ],
        out_specs=pl.BlockSpec((_BN, _D), lambda nb: (nb, 0)),
        out_shape=jax.ShapeDtypeStruct((_N, _D), jnp.float32),
    )(pre, h, nt2d, Astackl, b1ll, W2stackl, b2l)


def _sc_edge_agg(trw_flat, src2, dst2, et2, ntp):
    mesh = plsc.VectorSubcoreMesh(core_axis_name="c", subcore_axis_name="s")

    @functools.partial(
        pl.kernel,
        mesh=mesh,
        out_type=jax.ShapeDtypeStruct((2, _NPAD, _H), jnp.float32),
        scratch_types=[
            pltpu.VMEM((_NB_E, _SB), jnp.int32),
            pltpu.VMEM((_NB_E, _SB), jnp.int32),
            pltpu.VMEM((_NB_E, _SB), jnp.int32),
            pltpu.VMEM((_NB_E, _SB), jnp.int32),
            pltpu.VMEM((_NPAD,), jnp.int32),
            pltpu.VMEM((_SB, _H), jnp.float32),
            pltpu.VMEM_SHARED((_NPAD, _H), jnp.float32),
            pltpu.SemaphoreType.DMA,
        ],
    )
    def k(trw_hbm, src_hbm, dst_hbm, et_hbm, nt_hbm, out_hbm,
          src_v, dst_v, et_v, gi_v, nt_v, rows_v, acc, sem):
        c = lax.axis_index("c")
        s = lax.axis_index("s")
        wid = c * 16 + s
        ebase = wid * _NB_E

        pltpu.sync_copy(src_hbm.at[pl.ds(ebase, _NB_E)], src_v)
        pltpu.sync_copy(dst_hbm.at[pl.ds(ebase, _NB_E)], dst_v)
        pltpu.sync_copy(et_hbm.at[pl.ds(ebase, _NB_E)], et_v)
        pltpu.sync_copy(nt_hbm, nt_v)

        def _zrow(i, carry):
            for kk in range(8):
                rows_v[i, pl.ds(kk * 16, 16)] = jnp.zeros((16,), jnp.float32)
            return carry

        lax.fori_loop(0, _SB, _zrow, 0)
        for off, cnt in ((0, 128), (128, 128), (256, 128), (384, 128), (512, 114)):
            pltpu.sync_copy(
                rows_v.at[pl.ds(0, cnt)], acc.at[pl.ds(s * _RPT + off, cnt)]
            )

        def _gi_row(j, carry):
            for kk in range(8):
                sl = pl.ds(kk * 16, 16)
                d16 = dst_v[j, sl]
                nt16 = plsc.load_gather(nt_v, [d16])
                gi_v[j, sl] = (et_v[j, sl] * _N + src_v[j, sl]) * _T + nt16
            return carry

        lax.fori_loop(0, _NB_E, _gi_row, 0)

        plsc.subcore_barrier()

        def _edge_batch(j, carry):
            pltpu.async_copy(trw_hbm.at[gi_v.at[j]], rows_v, sem).wait()
            pltpu.sync_copy(rows_v, acc.at[dst_v.at[j]], add=True)
            return carry

        lax.fori_loop(0, _NB_E, _edge_batch, 0)

        plsc.subcore_barrier()
        pltpu.sync_copy(
            acc.at[pl.ds(s * _RPT, _RPT)], out_hbm.at[c, pl.ds(s * _RPT, _RPT)]
        )

    return k(trw_flat, src2, dst2, et2, ntp)


def kernel(node_feature, edge_index, edge_type, node_type,
           update_node_type_indices, update_edge_type_indices,
           rel_W1, rel_b1, rel_W2, rel_b2,
           node_W1, node_b1, node_W2, node_b2):
    src, dst = edge_index[0], edge_index[1]

    nw = node_W1.reshape(_L, _T, _R + 1, _D, _H)
    Astack = jnp.transpose(nw[:, :, 0], (0, 2, 1, 3)).reshape(_L, _D, _TH)
    Bstack = jnp.transpose(nw[:, :, 1:], (0, 2, 3, 1, 4)).reshape(_L, _R, _D, _TH)
    W2stack = node_W2.reshape(_L, _TH, _D)
    b1l = node_b1.reshape(_L, 1, _TH)
    rel_b1r = rel_b1.reshape(_L, _R, 1, _H)
    rel_b2r = rel_b2.reshape(_L, _R, 1, _H)

    C, dvec = _prep(rel_W2, Bstack, rel_b2r)

    pad = _EPAD - _E
    src2 = jnp.concatenate([src, jnp.zeros((pad,), jnp.int32)]).reshape(_EROWS, _SB)
    dst2 = jnp.concatenate([dst, jnp.full((pad,), _N, jnp.int32)]).reshape(_EROWS, _SB)
    et2 = jnp.concatenate([edge_type, jnp.zeros((pad,), jnp.int32)]).reshape(_EROWS, _SB)
    ntp = jnp.concatenate([node_type, jnp.zeros((_NPAD - _N,), jnp.int32)])
    nt2d = node_type.reshape(_N, 1)

    h = node_feature
    for l in range(_L):
        trw = _table(h, rel_W1[l], rel_b1r[l], C[l], dvec[l])
        trw_flat = trw.reshape(_R * _N * _T, _H)
        pre = _sc_edge_agg(trw_flat, src2, dst2, et2, ntp)
        h = _update(pre[:, :_N], h, nt2d, Astack[l], b1l[l], W2stack[l], node_b2[l])
    return h


# speedup vs baseline: 6.2600x; 6.2600x over previous
import functools

import jax
import jax.numpy as jnp
from jax import lax
from jax.experimental import pallas as pl
from jax.experimental.pallas import tpu as pltpu
from jax.experimental.pallas import tpu_sc as plsc

_N, _E, _L, _R, _T, _D, _H = 10000, 320000, 2, 4, 3, 128, 128
_TH = _T * _H
_BN = 1000
_NB = _N // _BN
_NTILES = 32
_SB = 128
_NB_E = 80
_EPT = _NB_E * _SB
_EPAD = _NTILES * _EPT
_EROWS = _EPAD // _SB
_NPAD = 10112
_RPT = _NPAD // 16
_GRP = 8


def _prep_body(w2_ref, bst_ref, b2_ref, c_ref, dvec_ref):
    b = bst_ref[0, 0]
    c_ref[0, 0] = jnp.dot(w2_ref[0, 0], b, preferred_element_type=jnp.float32)
    dvec_ref[0, 0] = jnp.dot(b2_ref[0, 0], b, preferred_element_type=jnp.float32)


def _prep(rel_W2, Bstack, rel_b2r):
    return pl.pallas_call(
        _prep_body,
        grid=(_L, _R),
        in_specs=[
            pl.BlockSpec((1, 1, _H, _H), lambda l, r: (l, r, 0, 0)),
            pl.BlockSpec((1, 1, _D, _TH), lambda l, r: (l, r, 0, 0)),
            pl.BlockSpec((1, 1, 1, _H), lambda l, r: (l, r, 0, 0)),
        ],
        out_specs=[
            pl.BlockSpec((1, 1, _H, _TH), lambda l, r: (l, r, 0, 0)),
            pl.BlockSpec((1, 1, 1, _TH), lambda l, r: (l, r, 0, 0)),
        ],
        out_shape=[
            jax.ShapeDtypeStruct((_L, _R, _H, _TH), jnp.float32),
            jax.ShapeDtypeStruct((_L, _R, 1, _TH), jnp.float32),
        ],
    )(rel_W2, Bstack, rel_b2r)


def _table_body(h_ref, w1_ref, b1_ref, c_ref, dvec_ref, out_ref):
    p = jax.nn.relu(
        jnp.dot(h_ref[...], w1_ref[0], preferred_element_type=jnp.float32)
        + b1_ref[0]
    )
    out_ref[0] = (
        jnp.dot(p, c_ref[0], preferred_element_type=jnp.float32) + dvec_ref[0]
    )


def _table(h, W1l, b1ll, Cl, dvecl):
    return pl.pallas_call(
        _table_body,
        grid=(_R, _NB),
        in_specs=[
            pl.BlockSpec((_BN, _D), lambda r, nb: (nb, 0)),
            pl.BlockSpec((1, _D, _H), lambda r, nb: (r, 0, 0)),
            pl.BlockSpec((1, 1, _H), lambda r, nb: (r, 0, 0)),
            pl.BlockSpec((1, _H, _TH), lambda r, nb: (r, 0, 0)),
            pl.BlockSpec((1, 1, _TH), lambda r, nb: (r, 0, 0)),
        ],
        out_specs=pl.BlockSpec((1, _BN, _TH), lambda r, nb: (r, nb, 0)),
        out_shape=jax.ShapeDtypeStruct((_R, _N, _TH), jnp.float32),
    )(h, W1l, b1ll, Cl, dvecl)


def _update_body(pre_ref, h_ref, nt_ref, a_ref, b1_ref, w2_ref, b2_ref, out_ref):
    s = pre_ref[0] + pre_ref[1]
    y = (
        jnp.dot(h_ref[...], a_ref[...], preferred_element_type=jnp.float32)
        + b1_ref[...]
    )
    nt = nt_ref[...]
    parts = []
    for t in range(_T):
        m = (nt == t).astype(jnp.float32)
        zt = jax.nn.relu(y[:, t * _H:(t + 1) * _H] + s)
        parts.append(zt * m)
    zcat = jnp.concatenate(parts, axis=1)
    out = jnp.dot(zcat, w2_ref[...], preferred_element_type=jnp.float32)
    for t in range(_T):
        m = (nt == t).astype(jnp.float32)
        out = out + m * b2_ref[t][None, :]
    out_ref[...] = out


def _update(pre, h, nt2d, Astackl, b1ll, W2stackl, b2l):
    return pl.pallas_call(
        _update_body,
        grid=(_NB,),
        in_specs=[
            pl.BlockSpec((2, _BN, _H), lambda nb: (0, nb, 0)),
            pl.BlockSpec((_BN, _D), lambda nb: (nb, 0)),
            pl.BlockSpec((_BN, 1), lambda nb: (nb, 0)),
            pl.BlockSpec((_D, _TH), lambda nb: (0, 0)),
            pl.BlockSpec((1, _TH), lambda nb: (0, 0)),
            pl.BlockSpec((_TH, _D), lambda nb: (0, 0)),
            pl.BlockSpec((_T, _H), lambda nb: (0, 0)),
        ],
        out_specs=pl.BlockSpec((_BN, _D), lambda nb: (nb, 0)),
        out_shape=jax.ShapeDtypeStruct((_N, _D), jnp.float32),
    )(pre, h, nt2d, Astackl, b1ll, W2stackl, b2l)


def _sc_edge_agg(trw_flat, src2, dst2, et2, ntp):
    mesh = plsc.VectorSubcoreMesh(core_axis_name="c", subcore_axis_name="s")

    @functools.partial(
        pl.kernel,
        mesh=mesh,
        out_type=jax.ShapeDtypeStruct((2, _NPAD, _H), jnp.float32),
        compiler_params=pltpu.CompilerParams(needs_layout_passes=False),
        scratch_types=[
            pltpu.VMEM((_GRP, _SB), jnp.int32),
            pltpu.VMEM((_GRP, _SB), jnp.int32),
            pltpu.VMEM((_GRP, _SB), jnp.int32),
            pltpu.VMEM((_GRP, _SB), jnp.int32),
            pltpu.VMEM((_NPAD,), jnp.int32),
            pltpu.VMEM((_SB, _H), jnp.float32),
            pltpu.VMEM_SHARED((_NPAD, _H), jnp.float32),
            pltpu.SemaphoreType.DMA,
        ],
    )
    def k(trw_hbm, src_hbm, dst_hbm, et_hbm, nt_hbm, out_hbm,
          src_g, dst_g, et_g, gi_g, nt_v, rows_v, acc, sem):
        c = lax.axis_index("c")
        s = lax.axis_index("s")
        wid = c * 16 + s

        pltpu.sync_copy(nt_hbm, nt_v)

        def _zrow(i, carry):
            for kk in range(8):
                rows_v[i, pl.ds(kk * 16, 16)] = jnp.zeros((16,), jnp.float32)
            return carry

        lax.fori_loop(0, _SB, _zrow, 0)
        for off, cnt in ((0, 128), (128, 128), (256, 128), (384, 128), (512, 120)):
            pltpu.sync_copy(
                rows_v.at[pl.ds(0, cnt)], acc.at[pl.ds(s * _RPT + off, cnt)]
            )
        plsc.subcore_barrier()

        def _group(g, carry):
            row0 = wid * _NB_E + g * _GRP
            pltpu.sync_copy(src_hbm.at[pl.ds(row0, _GRP)], src_g)
            pltpu.sync_copy(dst_hbm.at[pl.ds(row0, _GRP)], dst_g)
            pltpu.sync_copy(et_hbm.at[pl.ds(row0, _GRP)], et_g)
            for j in range(_GRP):
                for kk in range(8):
                    sl = pl.ds(kk * 16, 16)
                    d16 = dst_g[j, sl]
                    nt16 = plsc.load_gather(nt_v, [d16])
                    gi_g[j, sl] = (et_g[j, sl] * _N + src_g[j, sl]) * _T + nt16
            for j in range(_GRP):
                pltpu.async_copy(trw_hbm.at[gi_g.at[j]], rows_v, sem).wait()
                pltpu.sync_copy(rows_v, acc.at[dst_g.at[j]], add=True)
            return carry

        lax.fori_loop(0, _NB_E // _GRP, _group, 0)

        plsc.subcore_barrier()
        pltpu.sync_copy(
            acc.at[pl.ds(s * _RPT, _RPT)], out_hbm.at[c, pl.ds(s * _RPT, _RPT)]
        )

    return k(trw_flat, src2, dst2, et2, ntp)


def kernel(node_feature, edge_index, edge_type, node_type,
           update_node_type_indices, update_edge_type_indices,
           rel_W1, rel_b1, rel_W2, rel_b2,
           node_W1, node_b1, node_W2, node_b2):
    src, dst = edge_index[0], edge_index[1]

    nw = node_W1.reshape(_L, _T, _R + 1, _D, _H)
    Astack = jnp.transpose(nw[:, :, 0], (0, 2, 1, 3)).reshape(_L, _D, _TH)
    Bstack = jnp.transpose(nw[:, :, 1:], (0, 2, 3, 1, 4)).reshape(_L, _R, _D, _TH)
    W2stack = node_W2.reshape(_L, _TH, _D)
    b1l = node_b1.reshape(_L, 1, _TH)
    rel_b1r = rel_b1.reshape(_L, _R, 1, _H)
    rel_b2r = rel_b2.reshape(_L, _R, 1, _H)

    C, dvec = _prep(rel_W2, Bstack, rel_b2r)

    pad = _EPAD - _E
    src2 = jnp.concatenate([src, jnp.zeros((pad,), jnp.int32)]).reshape(_EROWS, _SB)
    dst2 = jnp.concatenate([dst, jnp.full((pad,), _N, jnp.int32)]).reshape(_EROWS, _SB)
    et2 = jnp.concatenate([edge_type, jnp.zeros((pad,), jnp.int32)]).reshape(_EROWS, _SB)
    ntp = jnp.concatenate([node_type, jnp.zeros((_NPAD - _N,), jnp.int32)])
    nt2d = node_type.reshape(_N, 1)

    h = node_feature
    for l in range(_L):
        trw = _table(h, rel_W1[l], rel_b1r[l], C[l], dvec[l])
        trw_flat = trw.reshape(_R * _N * _T, _H)
        pre = _sc_edge_agg(trw_flat, src2, dst2, et2, ntp)
        h = _update(pre[:, :_N], h, nt2d, Astack[l], b1l[l], W2stack[l], node_b2[l])
    return h
